# baseline (speedup 1.0000x reference)
import functools

import numpy as np
import jax
import jax.numpy as jnp
from jax import lax
from jax.experimental import pallas as pl
from jax.experimental.pallas import tpu as pltpu
from jax.experimental.pallas import tpu_sc as plsc

T = 20000
TP = 20480
D3 = 512
P1, P2 = 100003, 1009
OFF = P1 + P2 + 1

_OFFSETS = np.array(
    [[-1, 0, 0], [1, 0, 0], [0, -1, 0], [0, 1, 0], [0, 0, -1], [0, 0, 1],
     [-1, -1, 0], [-1, 1, 0], [1, -1, 0], [1, 1, 0], [-1, 0, -1], [-1, 0, 1],
     [1, 0, -1], [1, 0, 1], [0, -1, -1], [0, -1, 1], [0, 1, -1], [0, 1, 1],
     [-1, -1, -1], [-1, -1, 1], [-1, 1, -1], [-1, 1, 1], [1, -1, -1],
     [1, -1, 1], [1, 1, -1], [1, 1, 1]], dtype=np.int64)
DELTA = [int(o[0] * P1 + o[1] * P2 + o[2]) for o in _OFFSETS]
NK = 26

PROBE_MAX = 13030678
REGION = 6553600
WSLICE = 409600
ZB = 40960
SPLIT = 6515200
SHIFT = REGION - SPLIT
TOTAL = 2 * REGION
DUMMY0 = SPLIT
DUMMY1 = 13069080

CPW_A = TP // 16
CPW_B = TP // 32
RD_CHUNK = 2 * CPW_B
NRD = CPW_B * NK // RD_CHUNK

_lin = np.linspace(0.0, 1.0, 8)
_gx, _gy, _gz = np.meshgrid(_lin, _lin, _lin, indexing="ij")
_fdf = np.stack([_gx, 1.0 - _gx, _gy, 1.0 - _gy, _gz, 1.0 - _gz],
                axis=-1).reshape(-1, 6).astype(np.float32)
_PRI = np.empty((64, D3), np.float32)
for _m in range(64):
    _bits = np.array([(_m >> _f) & 1 for _f in range(6)], bool)
    _PRI[_m] = np.where(_bits[None, :], 1.0, _fdf).min(axis=-1)

_MESH = dict(core_axis_name="c", subcore_axis_name="s")


def _iota16():
    return lax.iota(jnp.int32, 16)


def _scatter_body(xs, ys, zs, table, zbuf, cx, cy, cz, sidx, svals, sem):
    c = lax.axis_index("c")
    s = lax.axis_index("s")

    def _zb(j, _):
        zbuf[pl.ds(j * 16, 16)] = jnp.zeros((16,), jnp.float32)
        return 0
    lax.fori_loop(0, ZB // 16, _zb, 0)
    ones = jnp.ones((16,), jnp.float32)
    for j in range(10):
        for t in range(8):
            svals[j, pl.ds(t * 16, 16)] = ones

    base_t = s * CPW_A
    pltpu.sync_copy(xs.at[pl.ds(pl.multiple_of(base_t, 8), CPW_A)], cx)
    pltpu.sync_copy(ys.at[pl.ds(pl.multiple_of(base_t, 8), CPW_A)], cy)
    pltpu.sync_copy(zs.at[pl.ds(pl.multiple_of(base_t, 8), CPW_A)], cz)

    lane = _iota16()
    lo = c * REGION
    dummy0 = jnp.where(c == 0, DUMMY0, DUMMY1) + s * CPW_A + lane
    for g in range(CPW_A // 16):
        xv = cx[pl.ds(g * 16, 16)]
        yv = cy[pl.ds(g * 16, 16)]
        zv = cz[pl.ds(g * 16, 16)]
        slot = xv * P1 + yv * P2 + zv + OFF
        phys = slot + jnp.where(slot >= SPLIT, SHIFT, 0)
        tid = base_t + g * 16 + lane
        ok = (tid < T) & (phys >= lo) & (phys < lo + REGION)
        final = jnp.where(ok, phys, dummy0 + g * 16)
        sidx[g // 8, pl.ds((g % 8) * 16, 16)] = final

    my_lo = c * REGION + s * WSLICE
    zcopies = [
        pltpu.async_copy(
            zbuf, table.at[pl.ds(pl.multiple_of(my_lo + j * ZB, 8), ZB)], sem)
        for j in range(WSLICE // ZB)]
    for cp in zcopies:
        cp.wait()

    plsc.subcore_barrier()

    copies = [pltpu.async_copy(svals.at[j], table.at[sidx.at[j]], sem)
              for j in range(10)]
    for cp in copies:
        cp.wait()


def _gather_body(xs, ys, zs, table, occkm,
                 cx, cy, cz, gidx, gvals, sem, sem2):
    c = lax.axis_index("c")
    s = lax.axis_index("s")
    wid = s * 2 + c
    base_t = wid * CPW_B

    pltpu.sync_copy(xs.at[pl.ds(pl.multiple_of(base_t, 8), CPW_B)], cx)
    pltpu.sync_copy(ys.at[pl.ds(pl.multiple_of(base_t, 8), CPW_B)], cy)
    pltpu.sync_copy(zs.at[pl.ds(pl.multiple_of(base_t, 8), CPW_B)], cz)

    def _build(i, _):
        xv = cx[pl.ds(i * 16, 16)]
        yv = cy[pl.ds(i * 16, 16)]
        zv = cz[pl.ds(i * 16, 16)]
        key = xv * P1 + yv * P2 + zv
        for k in range(NK):
            slot = key + (OFF + DELTA[k])
            phys = slot + jnp.where(slot >= SPLIT, SHIFT, 0)
            gidx[pl.ds(k * CPW_B + i * 16, 16)] = phys
        return 0
    lax.fori_loop(0, CPW_B // 16, _build, 0)

    rd = [pltpu.async_copy(table.at[gidx.at[pl.ds(j * RD_CHUNK, RD_CHUNK)]],
                           gvals.at[pl.ds(j * RD_CHUNK, RD_CHUNK)], sem)
          for j in range(NRD)]
    for cp in rd:
        cp.wait()

    wr = [pltpu.async_copy(gvals.at[pl.ds(k * CPW_B + j * 128, 128)],
                           occkm.at[5 * wid + j, k], sem2)
          for j in range(5) for k in range(NK)]
    for cp in wr:
        cp.wait()


def _make_sc_calls():
    scatter_call = functools.partial(
        pl.kernel,
        out_type=jax.ShapeDtypeStruct((TOTAL,), jnp.float32),
        mesh=plsc.VectorSubcoreMesh(**_MESH),
        scratch_types=[
            pltpu.VMEM((ZB,), jnp.float32),
            pltpu.VMEM((CPW_A,), jnp.int32),
            pltpu.VMEM((CPW_A,), jnp.int32),
            pltpu.VMEM((CPW_A,), jnp.int32),
            pltpu.VMEM((10, 128), jnp.int32),
            pltpu.VMEM((10, 128), jnp.float32),
            pltpu.SemaphoreType.DMA,
        ],
    )
    gather_call = functools.partial(
        pl.kernel,
        out_type=jax.ShapeDtypeStruct((TP // 128, NK, 128), jnp.float32),
        mesh=plsc.VectorSubcoreMesh(**_MESH),
        scratch_types=[
            pltpu.VMEM((CPW_B,), jnp.int32),
            pltpu.VMEM((CPW_B,), jnp.int32),
            pltpu.VMEM((CPW_B,), jnp.int32),
            pltpu.VMEM((CPW_B * NK,), jnp.int32),
            pltpu.VMEM((CPW_B * NK,), jnp.float32),
            pltpu.SemaphoreType.DMA,
            pltpu.SemaphoreType.DMA,
        ],
    )
    return scatter_call, gather_call

BT = 128


def _tc_body(x_ref, occkm_ref, pri_ref, w_ref, b_ref,
             out_ref, prior_ref, occ_ref):
    occT = occkm_ref[0].T
    occ_ref[...] = occT
    code = sum((occT[:, f:f + 1] * float(1 << f) for f in range(6)),
               jnp.zeros((BT, 1), jnp.float32)).astype(jnp.int32)
    iota = lax.broadcasted_iota(jnp.int32, (BT, 64), 1)
    oh = (code == iota).astype(jnp.float32)
    prior = jnp.dot(oh, pri_ref[...], preferred_element_type=jnp.float32)
    prior_ref[...] = prior
    out_ref[...] = (
        jnp.dot(x_ref[...] * prior, w_ref[...],
                preferred_element_type=jnp.float32) + b_ref[...])


def kernel(coords, x, W, b):
    ci = coords.astype(jnp.int32)
    pad = (0, TP - T)
    xs = jnp.pad(ci[:, 1], pad)
    ys = jnp.pad(ci[:, 2], pad)
    zs = jnp.pad(ci[:, 3], pad)

    scatter_call, gather_call = _make_sc_calls()
    table = scatter_call(_scatter_body)(xs, ys, zs)
    occkm = gather_call(_gather_body)(xs, ys, zs, table)

    pri = jnp.asarray(_PRI)
    out, prior, occ = pl.pallas_call(
        _tc_body,
        grid=((T + BT - 1) // BT,),
        in_specs=[
            pl.BlockSpec((BT, D3), lambda i: (i, 0)),
            pl.BlockSpec((1, NK, BT), lambda i: (i, 0, 0)),
            pl.BlockSpec((64, D3), lambda i: (0, 0)),
            pl.BlockSpec((D3, 256), lambda i: (0, 0)),
            pl.BlockSpec((1, 256), lambda i: (0, 0)),
        ],
        out_specs=[
            pl.BlockSpec((BT, 256), lambda i: (i, 0)),
            pl.BlockSpec((BT, D3), lambda i: (i, 0)),
            pl.BlockSpec((BT, NK), lambda i: (i, 0)),
        ],
        out_shape=[
            jax.ShapeDtypeStruct((T, 256), jnp.float32),
            jax.ShapeDtypeStruct((T, D3), jnp.float32),
            jax.ShapeDtypeStruct((T, NK), jnp.float32),
        ],
    )(x, occkm, pri, W, b.reshape(1, 256))
    return out, prior, occ


# speedup vs baseline: 1.2961x; 1.2961x over previous
import functools

import numpy as np
import jax
import jax.numpy as jnp
from jax import lax
from jax.experimental import pallas as pl
from jax.experimental.pallas import tpu as pltpu
from jax.experimental.pallas import tpu_sc as plsc

T = 20000
TP = 20480
D3 = 512
P1, P2 = 100003, 1009
OFF = P1 + P2 + 1

_OFFSETS = np.array(
    [[-1, 0, 0], [1, 0, 0], [0, -1, 0], [0, 1, 0], [0, 0, -1], [0, 0, 1],
     [-1, -1, 0], [-1, 1, 0], [1, -1, 0], [1, 1, 0], [-1, 0, -1], [-1, 0, 1],
     [1, 0, -1], [1, 0, 1], [0, -1, -1], [0, -1, 1], [0, 1, -1], [0, 1, 1],
     [-1, -1, -1], [-1, -1, 1], [-1, 1, -1], [-1, 1, 1], [1, -1, -1],
     [1, -1, 1], [1, 1, -1], [1, 1, 1]], dtype=np.int64)
DELTA = [int(o[0] * P1 + o[1] * P2 + o[2]) for o in _OFFSETS]
NK = 26

PROBE_MAX = 13030678
REGION = 6553600
WSLICE = 409600
ZB = 40960
SPLIT = 6515200
SHIFT = REGION - SPLIT
TOTAL = 2 * REGION
DUMMY0 = SPLIT
DUMMY1 = 13069080

CPW_A = TP // 16
CPW_B = TP // 32
RD_CHUNK = 2 * CPW_B
NRD = CPW_B * NK // RD_CHUNK

_lin = np.linspace(0.0, 1.0, 8)
_gx, _gy, _gz = np.meshgrid(_lin, _lin, _lin, indexing="ij")
_fdf = np.stack([_gx, 1.0 - _gx, _gy, 1.0 - _gy, _gz, 1.0 - _gz],
                axis=-1).reshape(-1, 6).astype(np.float32)
_PRI = np.empty((64, D3), np.float32)
for _m in range(64):
    _bits = np.array([(_m >> _f) & 1 for _f in range(6)], bool)
    _PRI[_m] = np.where(_bits[None, :], 1.0, _fdf).min(axis=-1)

_MESH = dict(core_axis_name="c", subcore_axis_name="s")


def _iota16():
    return lax.iota(jnp.int32, 16)


def _scatter_body(xs, ys, zs, zrow, table, zbuf, cx, cy, cz, sidx, svals,
                  sem, zsem):
    c = lax.axis_index("c")
    s = lax.axis_index("s")

    pltpu.sync_copy(zrow, zbuf)
    my_lo = c * REGION + s * WSLICE
    zcopies = [
        pltpu.async_copy(
            zbuf, table.at[pl.ds(pl.multiple_of(my_lo + j * ZB, 8), ZB)], zsem)
        for j in range(WSLICE // ZB)]

    ones = jnp.ones((16,), jnp.float32)
    for j in range(10):
        for t in range(8):
            svals[j, pl.ds(t * 16, 16)] = ones

    base_t = s * CPW_A
    pltpu.sync_copy(xs.at[pl.ds(pl.multiple_of(base_t, 8), CPW_A)], cx)
    pltpu.sync_copy(ys.at[pl.ds(pl.multiple_of(base_t, 8), CPW_A)], cy)
    pltpu.sync_copy(zs.at[pl.ds(pl.multiple_of(base_t, 8), CPW_A)], cz)

    lane = _iota16()
    lo = c * REGION
    dummy0 = jnp.where(c == 0, DUMMY0, DUMMY1) + s * CPW_A + lane
    for g in range(CPW_A // 16):
        xv = cx[pl.ds(g * 16, 16)]
        yv = cy[pl.ds(g * 16, 16)]
        zv = cz[pl.ds(g * 16, 16)]
        slot = xv * P1 + yv * P2 + zv + OFF
        phys = slot + jnp.where(slot >= SPLIT, SHIFT, 0)
        tid = base_t + g * 16 + lane
        ok = (tid < T) & (phys >= lo) & (phys < lo + REGION)
        final = jnp.where(ok, phys, dummy0 + g * 16)
        sidx[g // 8, pl.ds((g % 8) * 16, 16)] = final

    for cp in zcopies:
        cp.wait()

    plsc.subcore_barrier()

    copies = [pltpu.async_copy(svals.at[j], table.at[sidx.at[j]], sem)
              for j in range(10)]
    for cp in copies:
        cp.wait()


def _gather_body(xs, ys, zs, table, occkm, code_out,
                 cx, cy, cz, gidx, gvals, codes, sem, sem2):
    c = lax.axis_index("c")
    s = lax.axis_index("s")
    wid = s * 2 + c
    base_t = wid * CPW_B

    pltpu.sync_copy(xs.at[pl.ds(pl.multiple_of(base_t, 8), CPW_B)], cx)
    pltpu.sync_copy(ys.at[pl.ds(pl.multiple_of(base_t, 8), CPW_B)], cy)
    pltpu.sync_copy(zs.at[pl.ds(pl.multiple_of(base_t, 8), CPW_B)], cz)

    def _build(i, _):
        xv = cx[pl.ds(i * 16, 16)]
        yv = cy[pl.ds(i * 16, 16)]
        zv = cz[pl.ds(i * 16, 16)]
        key = xv * P1 + yv * P2 + zv
        for k in range(NK):
            slot = key + (OFF + DELTA[k])
            phys = slot + jnp.where(slot >= SPLIT, SHIFT, 0)
            gidx[pl.ds(k * CPW_B + i * 16, 16)] = phys
        return 0
    lax.fori_loop(0, CPW_B // 16, _build, 0)

    rd = [pltpu.async_copy(table.at[gidx.at[pl.ds(j * RD_CHUNK, RD_CHUNK)]],
                           gvals.at[pl.ds(j * RD_CHUNK, RD_CHUNK)], sem)
          for j in range(NRD)]
    for cp in rd:
        cp.wait()

    wr = [pltpu.async_copy(gvals.at[pl.ds(k * CPW_B + j * 128, 128)],
                           occkm.at[5 * wid + j, k], sem2)
          for j in range(5) for k in range(NK)]

    def _code(i, _):
        acc = jnp.zeros((16,), jnp.float32)
        for f in range(6):
            acc = acc + gvals[pl.ds(f * CPW_B + i * 16, 16)] * float(1 << f)
        codes[pl.ds(i * 16, 16)] = acc
        return 0
    lax.fori_loop(0, CPW_B // 16, _code, 0)
    pltpu.sync_copy(codes, code_out.at[pl.ds(pl.multiple_of(base_t, 8), CPW_B)])

    for cp in wr:
        cp.wait()


def _make_sc_calls():
    scatter_call = functools.partial(
        pl.kernel,
        out_type=jax.ShapeDtypeStruct((TOTAL,), jnp.float32),
        mesh=plsc.VectorSubcoreMesh(**_MESH),
        scratch_types=[
            pltpu.VMEM((ZB,), jnp.float32),
            pltpu.VMEM((CPW_A,), jnp.int32),
            pltpu.VMEM((CPW_A,), jnp.int32),
            pltpu.VMEM((CPW_A,), jnp.int32),
            pltpu.VMEM((10, 128), jnp.int32),
            pltpu.VMEM((10, 128), jnp.float32),
            pltpu.SemaphoreType.DMA,
            pltpu.SemaphoreType.DMA,
        ],
    )
    gather_call = functools.partial(
        pl.kernel,
        out_type=(jax.ShapeDtypeStruct((TP // 128, NK, 128), jnp.float32),
                  jax.ShapeDtypeStruct((TP,), jnp.float32)),
        mesh=plsc.VectorSubcoreMesh(**_MESH),
        scratch_types=[
            pltpu.VMEM((CPW_B,), jnp.int32),
            pltpu.VMEM((CPW_B,), jnp.int32),
            pltpu.VMEM((CPW_B,), jnp.int32),
            pltpu.VMEM((CPW_B * NK,), jnp.int32),
            pltpu.VMEM((CPW_B * NK,), jnp.float32),
            pltpu.VMEM((CPW_B,), jnp.float32),
            pltpu.SemaphoreType.DMA,
            pltpu.SemaphoreType.DMA,
        ],
    )
    return scatter_call, gather_call

BT = 1000
OB = 8


def _occ_body(occkm_ref, occ_ref):
    occ_ref[...] = jnp.concatenate(
        [occkm_ref[j].T for j in range(OB)], axis=0)


def _tc_body(x_ref, code_ref, pri_ref, w_ref, b_ref, out_ref, prior_ref):
    code = code_ref[...].astype(jnp.int32)
    iota = lax.broadcasted_iota(jnp.int32, (BT, 64), 1)
    oh = (code == iota).astype(jnp.float32)
    prior = jnp.dot(oh, pri_ref[...], preferred_element_type=jnp.float32)
    prior_ref[...] = prior
    out_ref[...] = (
        jnp.dot(x_ref[...] * prior, w_ref[...],
                preferred_element_type=jnp.float32) + b_ref[...])


def kernel(coords, x, W, b):
    ci = coords.astype(jnp.int32)
    pad = (0, TP - T)
    xs = jnp.pad(ci[:, 1], pad)
    ys = jnp.pad(ci[:, 2], pad)
    zs = jnp.pad(ci[:, 3], pad)

    scatter_call, gather_call = _make_sc_calls()
    zrow = jnp.zeros((ZB,), jnp.float32)
    table = scatter_call(_scatter_body)(xs, ys, zs, zrow)
    occkm, code = gather_call(_gather_body)(xs, ys, zs, table)
    code2d = code[:T].reshape(T, 1)

    occ = pl.pallas_call(
        _occ_body,
        grid=(TP // (OB * 128),),
        in_specs=[pl.BlockSpec((OB, NK, 128), lambda i: (i, 0, 0))],
        out_specs=pl.BlockSpec((OB * 128, NK), lambda i: (i, 0)),
        out_shape=jax.ShapeDtypeStruct((T, NK), jnp.float32),
    )(occkm)

    pri = jnp.asarray(_PRI)
    out, prior = pl.pallas_call(
        _tc_body,
        grid=(T // BT,),
        in_specs=[
            pl.BlockSpec((BT, D3), lambda i: (i, 0)),
            pl.BlockSpec((BT, 1), lambda i: (i, 0)),
            pl.BlockSpec((64, D3), lambda i: (0, 0)),
            pl.BlockSpec((D3, 256), lambda i: (0, 0)),
            pl.BlockSpec((1, 256), lambda i: (0, 0)),
        ],
        out_specs=[
            pl.BlockSpec((BT, 256), lambda i: (i, 0)),
            pl.BlockSpec((BT, D3), lambda i: (i, 0)),
        ],
        out_shape=[
            jax.ShapeDtypeStruct((T, 256), jnp.float32),
            jax.ShapeDtypeStruct((T, D3), jnp.float32),
        ],
    )(x, code2d, pri, W, b.reshape(1, 256))
    return out, prior, occ
